# Initial kernel scaffold; baseline (speedup 1.0000x reference)
import jax
import jax.numpy as jnp
from jax.experimental import pallas as pl


def kernel(x, pos, pe_w, pe_b, rms_g, w_qkv, k_posemb, v_posemb, k_w1, k_w2, v_w1, v_w2, mem_k, mem_v, w_comb, b_comb, w_out):
    raise NotImplementedError("write your pallas kernel here")


# speedup vs baseline: 1.3666x; 1.3666x over previous
import functools

import jax
import jax.numpy as jnp
from jax.experimental import pallas as pl
from jax.experimental.pallas import tpu as pltpu

N_TOK = 4096; DIM = 1024; HEADS = 16; DH = 64; BALL = 256
WIN = 16; BC = 16; SC = 8; BF = 16
NB = N_TOK // BALL
NW = (BALL - BC) // SC + 1
HALF = DH // 2
NF = BALL // BF
SCALE = DH ** -0.5
F32 = jnp.float32


def _rope(t, cos, sin):
    t1, t2 = t[..., :HALF], t[..., HALF:]
    return jnp.concatenate([t1 * cos - t2 * sin, t1 * sin + t2 * cos], axis=-1)


def _prep_body(x_ref, pos_ref, pe_w_ref, pe_b_ref, rms_g_ref, w_qkv_ref,
               w_comb_ref, b_comb_ref, qkv_ref, gates_ref):
    posb = pos_ref[...]
    rel = posb - jnp.mean(posb, axis=0, keepdims=True)
    xb = x_ref[...] + jnp.dot(rel, pe_w_ref[...], preferred_element_type=F32) \
        + pe_b_ref[...]
    ms = jnp.mean(xb * xb, axis=-1, keepdims=True)
    xn = xb * jax.lax.rsqrt(ms + 1e-6) * rms_g_ref[...]
    qkv_ref[...] = jnp.dot(xn, w_qkv_ref[...], preferred_element_type=F32)
    gates_ref[...] = jax.nn.sigmoid(
        jnp.dot(xn, w_comb_ref[...], preferred_element_type=F32)
        + b_comb_ref[...])


def _comp_body(k_ref, v_ref, cos_ref, sin_ref, kpe_ref, vpe_ref,
               kw1_ref, kw2_ref, vw1_ref, vw2_ref, ck_ref, cv_ref):
    cos = cos_ref[...][None]
    sin = sin_ref[...][None]

    def branch(src, pe, w1, w2, out_ref, do_rope):
        t = src[:, 0]
        if do_rope:
            t = _rope(t, cos, sin)
        wins = [t[:, s0:s0 + BC, :] for s0 in range(0, SC * NW, SC)]
        tw = jnp.stack(wins, axis=1) + pe[0][None, None]
        flat = tw.reshape(NB * NW, BC * DH)
        h1 = jnp.maximum(jnp.dot(flat, w1[0], preferred_element_type=F32), 0.0)
        out = jnp.dot(h1, w2[0], preferred_element_type=F32)
        out_ref[0] = out.reshape(NB, NW, DH)

    branch(k_ref, kpe_ref, kw1_ref, kw2_ref, ck_ref, True)
    branch(v_ref, vpe_ref, vw1_ref, vw2_ref, cv_ref, False)


def _softmax(x):
    m = jnp.max(x, axis=-1, keepdims=True)
    e = jnp.exp(x - m)
    return e / jnp.sum(e, axis=-1, keepdims=True)


def _attn_body(q_ref, k_ref, v_ref, ck_ref, cv_ref, mk_ref, mv_ref,
               gates_ref, cos_ref, sin_ref, pool_ref, o_ref):
    cos = cos_ref[...]
    sin = sin_ref[...]
    q = _rope(q_ref[0, 0], cos, sin)
    k = _rope(k_ref[0, 0], cos, sin)
    v = v_ref[0, 0]

    ckf = jnp.concatenate([mk_ref[0], ck_ref[0, 0]], axis=0)
    cvf = jnp.concatenate([mv_ref[0], cv_ref[0, 0]], axis=0)
    csim = jax.lax.dot_general(q, ckf, (((1,), (1,)), ((), ())),
                               preferred_element_type=F32) * SCALE
    ci = jax.lax.broadcasted_iota(jnp.int32, (BALL, NW + 1), 0)
    cj = jax.lax.broadcasted_iota(jnp.int32, (BALL, NW + 1), 1)
    cvis = (cj == 0) | (SC * cj + BC - SC - 1 < ci)
    cattn = _softmax(jnp.where(cvis, csim, -1e10))
    c_out = jnp.dot(cattn, cvf, preferred_element_type=F32)

    pooled = jnp.dot(cattn, pool_ref[...], preferred_element_type=F32)
    pmax = jnp.max(pooled, axis=-1, keepdims=True)
    fidx = jax.lax.broadcasted_iota(jnp.int32, (BALL, NF), 1)
    sel = jnp.min(jnp.where(pooled == pmax, fidx, NF), axis=-1, keepdims=True)

    sim = jax.lax.dot_general(q, k, (((1,), (1,)), ((), ())),
                              preferred_element_type=F32) * SCALE
    ib = jax.lax.broadcasted_iota(jnp.int32, (BALL, BALL), 0)
    jb = jax.lax.broadcasted_iota(jnp.int32, (BALL, BALL), 1)
    causal = jb <= ib
    jblk = jb // BF
    fmask = causal & ((jblk == ib // BF) | (jblk == sel))
    f_out = jnp.dot(_softmax(jnp.where(fmask, sim, -1e10)), v,
                    preferred_element_type=F32)
    smask = causal & (ib - jb < WIN)
    s_out = jnp.dot(_softmax(jnp.where(smask, sim, -1e10)), v,
                    preferred_element_type=F32)

    g = gates_ref[0, 0]
    o_ref[0, 0] = (g[:, 0:1] * c_out + g[:, 1:2] * f_out + g[:, 2:3] * s_out)


def _proj_body(y_ref, w_ref, o_ref):
    o_ref[...] = jnp.dot(y_ref[...], w_ref[...], preferred_element_type=F32)


def kernel(x, pos, pe_w, pe_b, rms_g, w_qkv, k_posemb, v_posemb, k_w1, k_w2,
           v_w1, v_w2, mem_k, mem_v, w_comb, b_comb, w_out):
    freqs = 1.0 / (10000.0 ** (jnp.arange(HALF, dtype=F32) / HALF))
    ang = jnp.arange(BALL, dtype=F32)[:, None] * freqs[None, :]
    cosv, sinv = jnp.cos(ang), jnp.sin(ang)
    starts = jnp.arange(NW) * SC
    pool = jnp.concatenate(
        [jnp.zeros((1, NF), F32),
         jax.nn.one_hot(starts // BF, NF, dtype=F32)], axis=0)

    grid1 = (NB,)
    qkv, gates = pl.pallas_call(
        _prep_body,
        grid=grid1,
        in_specs=[
            pl.BlockSpec((BALL, DIM), lambda b: (b, 0)),
            pl.BlockSpec((BALL, 3), lambda b: (b, 0)),
            pl.BlockSpec((3, DIM), lambda b: (0, 0)),
            pl.BlockSpec((1, DIM), lambda b: (0, 0)),
            pl.BlockSpec((1, DIM), lambda b: (0, 0)),
            pl.BlockSpec((DIM, 3 * DIM), lambda b: (0, 0)),
            pl.BlockSpec((DIM, 3 * HEADS), lambda b: (0, 0)),
            pl.BlockSpec((1, 3 * HEADS), lambda b: (0, 0)),
        ],
        out_specs=[
            pl.BlockSpec((BALL, 3 * DIM), lambda b: (b, 0)),
            pl.BlockSpec((BALL, 3 * HEADS), lambda b: (b, 0)),
        ],
        out_shape=[
            jax.ShapeDtypeStruct((N_TOK, 3 * DIM), F32),
            jax.ShapeDtypeStruct((N_TOK, 3 * HEADS), F32),
        ],
    )(x, pos, pe_w, pe_b.reshape(1, DIM), rms_g.reshape(1, DIM), w_qkv,
      w_comb, b_comb.reshape(1, 3 * HEADS))

    qkv = qkv.reshape(NB, BALL, 3, HEADS, DH)
    q = qkv[:, :, 0].transpose(0, 2, 1, 3)
    k = qkv[:, :, 1].transpose(0, 2, 1, 3)
    v = qkv[:, :, 2].transpose(0, 2, 1, 3)
    gates = gates.reshape(NB, BALL, HEADS, 3).transpose(0, 2, 1, 3)

    ck, cv = pl.pallas_call(
        _comp_body,
        grid=(HEADS,),
        in_specs=[
            pl.BlockSpec((NB, 1, BALL, DH), lambda h: (0, h, 0, 0)),
            pl.BlockSpec((NB, 1, BALL, DH), lambda h: (0, h, 0, 0)),
            pl.BlockSpec((BALL, HALF), lambda h: (0, 0)),
            pl.BlockSpec((BALL, HALF), lambda h: (0, 0)),
            pl.BlockSpec((1, BC, DH), lambda h: (h, 0, 0)),
            pl.BlockSpec((1, BC, DH), lambda h: (h, 0, 0)),
            pl.BlockSpec((1, BC * DH, BC * DH), lambda h: (h, 0, 0)),
            pl.BlockSpec((1, BC * DH, DH), lambda h: (h, 0, 0)),
            pl.BlockSpec((1, BC * DH, BC * DH), lambda h: (h, 0, 0)),
            pl.BlockSpec((1, BC * DH, DH), lambda h: (h, 0, 0)),
        ],
        out_specs=[
            pl.BlockSpec((1, NB, NW, DH), lambda h: (h, 0, 0, 0)),
            pl.BlockSpec((1, NB, NW, DH), lambda h: (h, 0, 0, 0)),
        ],
        out_shape=[
            jax.ShapeDtypeStruct((HEADS, NB, NW, DH), F32),
            jax.ShapeDtypeStruct((HEADS, NB, NW, DH), F32),
        ],
    )(k, v, cosv, sinv, k_posemb, v_posemb, k_w1, k_w2, v_w1, v_w2)

    attn = pl.pallas_call(
        _attn_body,
        grid=(NB, HEADS),
        in_specs=[
            pl.BlockSpec((1, 1, BALL, DH), lambda b, h: (b, h, 0, 0)),
            pl.BlockSpec((1, 1, BALL, DH), lambda b, h: (b, h, 0, 0)),
            pl.BlockSpec((1, 1, BALL, DH), lambda b, h: (b, h, 0, 0)),
            pl.BlockSpec((1, 1, NW, DH), lambda b, h: (h, b, 0, 0)),
            pl.BlockSpec((1, 1, NW, DH), lambda b, h: (h, b, 0, 0)),
            pl.BlockSpec((1, 1, DH), lambda b, h: (h, 0, 0)),
            pl.BlockSpec((1, 1, DH), lambda b, h: (h, 0, 0)),
            pl.BlockSpec((1, 1, BALL, 3), lambda b, h: (b, h, 0, 0)),
            pl.BlockSpec((BALL, HALF), lambda b, h: (0, 0)),
            pl.BlockSpec((BALL, HALF), lambda b, h: (0, 0)),
            pl.BlockSpec((NW + 1, NF), lambda b, h: (0, 0)),
        ],
        out_specs=pl.BlockSpec((1, 1, BALL, DH), lambda b, h: (b, h, 0, 0)),
        out_shape=jax.ShapeDtypeStruct((NB, HEADS, BALL, DH), F32),
    )(q, k, v, ck, cv, mem_k, mem_v, gates, cosv, sinv, pool)

    y = attn.transpose(0, 2, 1, 3).reshape(N_TOK, HEADS * DH)
    out = pl.pallas_call(
        _proj_body,
        grid=(8,),
        in_specs=[
            pl.BlockSpec((N_TOK // 8, DIM), lambda i: (i, 0)),
            pl.BlockSpec((DIM, DIM), lambda i: (0, 0)),
        ],
        out_specs=pl.BlockSpec((N_TOK // 8, DIM), lambda i: (i, 0)),
        out_shape=jax.ShapeDtypeStruct((N_TOK, DIM), F32),
    )(y, w_out)
    return out
